# baseline (speedup 1.0000x reference)
import jax
import jax.numpy as jnp
from jax import lax
from jax.experimental import pallas as pl
from jax.experimental.pallas import tpu as pltpu
from jax.experimental.pallas import tpu_sc as plsc

PE = 8192
B = 4096
L = 16
NC, NS = 2, 16
NW = NC * NS
RW = B // NW
G = PE // L
GW = RW * G
EW = RW * PE
ZWORDS = 16384
NFIRE = EW // ZWORDS


def _body(pos_hbm, out_hbm, posv, zbuf, srcbuf, qbuf, semz, sems):
    wid = lax.axis_index("s") * NC + lax.axis_index("c")
    ebase = wid * RW * PE

    pltpu.sync_copy(pos_hbm.at[pl.ds(wid * RW, RW)], posv)

    zero16 = jnp.zeros((L,), jnp.float32)

    def zloop(i, carry):
        for j in range(8):
            zbuf[pl.ds((i * 8 + j) * L, L)] = zero16
        return carry

    lax.fori_loop(0, ZWORDS // (8 * L), zloop, 0)

    copies = []
    for j in range(NFIRE):
        copies.append(
            pltpu.async_copy(
                zbuf, out_hbm.at[pl.ds(ebase + j * ZWORDS, ZWORDS)], semz
            )
        )

    iota = lax.iota(jnp.int32, L)
    one16 = jnp.ones((L,), jnp.float32)
    for g in range(RW // L):
        pos16 = posv[pl.ds(g * L, L)]
        pos16 = jnp.minimum(jnp.maximum(pos16, 0), PE - 1)
        qbuf[pl.ds(g * L, L)] = (wid * RW + g * L + iota) * PE + pos16
        srcbuf[pl.ds(g * L, L)] = one16

    for c in copies:
        c.wait()
    pltpu.async_copy(srcbuf, out_hbm.at[qbuf], sems).wait()


@jax.jit
def _one_hot(position):
    mesh = plsc.VectorSubcoreMesh(core_axis_name="c", subcore_axis_name="s")
    out = pl.kernel(
        _body,
        out_type=jax.ShapeDtypeStruct((B * PE,), jnp.float32),
        mesh=mesh,
        scratch_types=[
            pltpu.VMEM((RW,), jnp.int32),
            pltpu.VMEM((ZWORDS,), jnp.float32),
            pltpu.VMEM((RW,), jnp.float32),
            pltpu.VMEM((RW,), jnp.int32),
            pltpu.SemaphoreType.DMA,
            pltpu.SemaphoreType.DMA,
        ],
    )(position)
    return out.reshape(B, PE)


def kernel(position):
    if position.ndim > 1 and position.shape[-1] == 1:
        position = jnp.squeeze(position, axis=-1)
    return _one_hot(position.astype(jnp.int32))


# speedup vs baseline: 4.8434x; 4.8434x over previous
import jax
import jax.numpy as jnp
from jax import lax
from jax.experimental import pallas as pl
from jax.experimental.pallas import tpu as pltpu

PE = 8192
B = 4096
BR = 256


def _body(pos_ref, out_ref):
    i = pl.program_id(0)
    p = pos_ref[pl.ds(i * BR, BR)]
    p = jnp.minimum(jnp.maximum(p, 0), PE - 1)
    col = lax.broadcasted_iota(jnp.int32, (BR, PE), 1)
    out_ref[...] = (col == p[:, None]).astype(jnp.float32)


@jax.jit
def _one_hot(position):
    return pl.pallas_call(
        _body,
        grid=(B // BR,),
        in_specs=[pl.BlockSpec((B,), lambda i: (0,))],
        out_specs=pl.BlockSpec((BR, PE), lambda i: (i, 0)),
        out_shape=jax.ShapeDtypeStruct((B, PE), jnp.float32),
    )(position)


def kernel(position):
    if position.ndim > 1 and position.shape[-1] == 1:
        position = jnp.squeeze(position, axis=-1)
    return _one_hot(position.astype(jnp.int32))
